# baseline (speedup 1.0000x reference)
import functools

import jax
import jax.numpy as jnp
from jax import lax
from jax.experimental import pallas as pl
from jax.experimental.pallas import tpu as pltpu
from jax.experimental.pallas import tpu_sc as plsc

N_NODES = 10000
N_EDGES = 320000
D_FEAT = 128
LANES = 16

NC = 2
NS = 16
NW = NC * NS

EDGES_PER_TILE = N_EDGES // NW
CHUNK = 80
NCHUNK = EDGES_PER_TILE // CHUNK
NGROUP = CHUNK // LANES


ROWS_PER_SUBCORE = (N_NODES // NS) // 8 * 8
ROWS_TAIL = N_NODES - NS * ROWS_PER_SUBCORE


def _body(src_hbm, dst_hbm, model_hbm, out_hbm,
          sidx, didx, srows, drows, outv, table, sem):
    cid = lax.axis_index("c")
    sid = lax.axis_index("s")
    wid = sid * NC + cid
    base = wid * EDGES_PER_TILE

    rb = sid * ROWS_PER_SUBCORE
    pltpu.sync_copy(model_hbm.at[pl.ds(rb, ROWS_PER_SUBCORE)],
                    table.at[pl.ds(rb, ROWS_PER_SUBCORE)])

    @pl.when(sid == NS - 1)
    def _stage_tail():
        tb = NS * ROWS_PER_SUBCORE
        pltpu.sync_copy(model_hbm.at[pl.ds(tb, ROWS_TAIL)],
                        table.at[pl.ds(tb, ROWS_TAIL)])

    plsc.subcore_barrier()

    def chunk_step(j, _):
        off = base + j * CHUNK
        pltpu.sync_copy(src_hbm.at[pl.ds(off, CHUNK)], sidx)
        pltpu.sync_copy(dst_hbm.at[pl.ds(off, CHUNK)], didx)
        g0 = pltpu.async_copy(table.at[sidx], srows, sem)
        g1 = pltpu.async_copy(table.at[didx], drows, sem)
        g0.wait()
        g1.wait()

        lane = lax.iota(jnp.int32, LANES)

        def group_step(g, _):
            rows = g * LANES + lane
            acc = jnp.zeros((LANES,), jnp.float32)
            for d in range(D_FEAT):
                col = jnp.full((LANES,), d, jnp.int32)
                acc += (plsc.load_gather(srows, [rows, col])
                        * plsc.load_gather(drows, [rows, col]))
            outv[pl.ds(g * LANES, LANES)] = acc
            return 0

        lax.fori_loop(0, NGROUP, group_step, 0)
        pltpu.sync_copy(outv, out_hbm.at[pl.ds(off, CHUNK)])
        return 0

    lax.fori_loop(0, NCHUNK, chunk_step, 0)


@jax.jit
def _run(src, dst, model):
    mesh = plsc.VectorSubcoreMesh(core_axis_name="c", subcore_axis_name="s")
    return pl.kernel(
        _body,
        out_type=jax.ShapeDtypeStruct((N_EDGES,), jnp.float32),
        mesh=mesh,
        compiler_params=pltpu.CompilerParams(needs_layout_passes=False),
        scratch_types=[
            pltpu.VMEM((CHUNK,), jnp.int32),
            pltpu.VMEM((CHUNK,), jnp.int32),
            pltpu.VMEM((CHUNK, D_FEAT), jnp.float32),
            pltpu.VMEM((CHUNK, D_FEAT), jnp.float32),
            pltpu.VMEM((CHUNK,), jnp.float32),
            pltpu.VMEM_SHARED((N_NODES, D_FEAT), jnp.float32),
            pltpu.SemaphoreType.DMA,
        ],
    )(src, dst, model)


def kernel(model, edge_index):
    ei = edge_index.astype(jnp.int32)
    return _run(ei[0], ei[1], model)


# speedup vs baseline: 1.0515x; 1.0515x over previous
import functools

import jax
import jax.numpy as jnp
from jax import lax
from jax.experimental import pallas as pl
from jax.experimental.pallas import tpu as pltpu
from jax.experimental.pallas import tpu_sc as plsc

N_NODES = 10000
N_EDGES = 320000
D_FEAT = 128
LANES = 16

NC = 2
NS = 16
NW = NC * NS

EDGES_PER_TILE = N_EDGES // NW
CHUNK = 400
NCHUNK = EDGES_PER_TILE // CHUNK
NGROUP = CHUNK // LANES


NSPLIT = 5
SUB = CHUNK // NSPLIT


def _body(src_hbm, dst_hbm, model_hbm, out_hbm,
          sidx, didx, srows, drows, outv, sem):
    cid = lax.axis_index("c")
    sid = lax.axis_index("s")
    wid = sid * NC + cid
    base = wid * EDGES_PER_TILE

    def chunk_step(j, _):
        off = base + j * CHUNK
        pltpu.sync_copy(src_hbm.at[pl.ds(off, CHUNK)], sidx)
        pltpu.sync_copy(dst_hbm.at[pl.ds(off, CHUNK)], didx)
        waits = []
        for k in range(NSPLIT):
            s = pl.ds(k * SUB, SUB)
            waits.append(pltpu.async_copy(
                model_hbm.at[sidx.at[s]], srows.at[s], sem))
            waits.append(pltpu.async_copy(
                model_hbm.at[didx.at[s]], drows.at[s], sem))
        for w in waits:
            w.wait()

        lane = lax.iota(jnp.int32, LANES)

        def group_step(g, _):
            rows = g * LANES + lane
            acc = jnp.zeros((LANES,), jnp.float32)
            for d in range(D_FEAT):
                col = jnp.full((LANES,), d, jnp.int32)
                acc += (plsc.load_gather(srows, [rows, col])
                        * plsc.load_gather(drows, [rows, col]))
            outv[pl.ds(g * LANES, LANES)] = acc
            return 0

        lax.fori_loop(0, NGROUP, group_step, 0)
        pltpu.sync_copy(outv, out_hbm.at[pl.ds(off, CHUNK)])
        return 0

    lax.fori_loop(0, NCHUNK, chunk_step, 0)


@jax.jit
def _run(src, dst, model):
    mesh = plsc.VectorSubcoreMesh(core_axis_name="c", subcore_axis_name="s")
    return pl.kernel(
        _body,
        out_type=jax.ShapeDtypeStruct((N_EDGES,), jnp.float32),
        mesh=mesh,
        compiler_params=pltpu.CompilerParams(needs_layout_passes=False),
        scratch_types=[
            pltpu.VMEM((CHUNK,), jnp.int32),
            pltpu.VMEM((CHUNK,), jnp.int32),
            pltpu.VMEM((CHUNK, D_FEAT), jnp.float32),
            pltpu.VMEM((CHUNK, D_FEAT), jnp.float32),
            pltpu.VMEM((CHUNK,), jnp.float32),
            pltpu.SemaphoreType.DMA,
        ],
    )(src, dst, model)


def kernel(model, edge_index):
    ei = edge_index.astype(jnp.int32)
    return _run(ei[0], ei[1], model)


# speedup vs baseline: 1.9083x; 1.8149x over previous
import functools

import jax
import jax.numpy as jnp
from jax import lax
from jax.experimental import pallas as pl
from jax.experimental.pallas import tpu as pltpu
from jax.experimental.pallas import tpu_sc as plsc

N_NODES = 10000
N_EDGES = 320000
D_FEAT = 128
D_PACK = D_FEAT // 2
LANES = 16

NC = 2
NS = 16
NW = NC * NS

EDGES_PER_TILE = N_EDGES // NW
CHUNK = 400
NCHUNK = EDGES_PER_TILE // CHUNK
NGROUP = CHUNK // LANES

NSPLIT = 5
SUB = CHUNK // NSPLIT


def _body(src_hbm, dst_hbm, model_hbm, out_hbm,
          sidx, didx, srows, drows, outv, sem):
    cid = lax.axis_index("c")
    sid = lax.axis_index("s")
    wid = sid * NC + cid
    base = wid * EDGES_PER_TILE

    def chunk_step(j, _):
        off = base + j * CHUNK
        pltpu.sync_copy(src_hbm.at[pl.ds(off, CHUNK)], sidx)
        pltpu.sync_copy(dst_hbm.at[pl.ds(off, CHUNK)], didx)
        waits = []
        for k in range(NSPLIT):
            s = pl.ds(k * SUB, SUB)
            waits.append(pltpu.async_copy(
                model_hbm.at[sidx.at[s]], srows.at[s], sem))
            waits.append(pltpu.async_copy(
                model_hbm.at[didx.at[s]], drows.at[s], sem))
        for w in waits:
            w.wait()

        lane = lax.iota(jnp.int32, LANES)

        def group_step(g, _):
            rows = g * LANES + lane
            acc = jnp.zeros((LANES,), jnp.float32)
            for d in range(D_PACK):
                col = jnp.full((LANES,), d, jnp.int32)
                ws = plsc.bitcast(plsc.load_gather(srows, [rows, col]),
                                  jnp.bfloat16)
                wd = plsc.bitcast(plsc.load_gather(drows, [rows, col]),
                                  jnp.bfloat16)
                s_lo, s_hi = plsc.unpack(ws, format=plsc.PackFormat.INTERLEAVED)
                d_lo, d_hi = plsc.unpack(wd, format=plsc.PackFormat.INTERLEAVED)
                acc += s_lo * d_lo
                acc += s_hi * d_hi
            outv[pl.ds(g * LANES, LANES)] = acc
            return 0

        lax.fori_loop(0, NGROUP, group_step, 0)
        pltpu.sync_copy(outv, out_hbm.at[pl.ds(off, CHUNK)])
        return 0

    lax.fori_loop(0, NCHUNK, chunk_step, 0)


@jax.jit
def _run(src, dst, model_packed):
    mesh = plsc.VectorSubcoreMesh(core_axis_name="c", subcore_axis_name="s")
    return pl.kernel(
        _body,
        out_type=jax.ShapeDtypeStruct((N_EDGES,), jnp.float32),
        mesh=mesh,
        compiler_params=pltpu.CompilerParams(needs_layout_passes=False,
                                             use_tc_tiling_on_sc=False),
        scratch_types=[
            pltpu.VMEM((CHUNK,), jnp.int32),
            pltpu.VMEM((CHUNK,), jnp.int32),
            pltpu.VMEM((CHUNK, D_PACK), jnp.int32),
            pltpu.VMEM((CHUNK, D_PACK), jnp.int32),
            pltpu.VMEM((CHUNK,), jnp.float32),
            pltpu.SemaphoreType.DMA,
        ],
    )(src, dst, model_packed)


def kernel(model, edge_index):
    ei = edge_index.astype(jnp.int32)
    mp = lax.bitcast_convert_type(
        model.astype(jnp.bfloat16).reshape(N_NODES, D_PACK, 2), jnp.int32)
    return _run(ei[0], ei[1], mp)


# speedup vs baseline: 2.0693x; 1.0844x over previous
import functools

import jax
import jax.numpy as jnp
from jax import lax
from jax.experimental import pallas as pl
from jax.experimental.pallas import tpu as pltpu
from jax.experimental.pallas import tpu_sc as plsc

N_NODES = 10000
N_EDGES = 320000
D_FEAT = 128
D_PACK = D_FEAT // 2
LANES = 16

NC = 2
NS = 16
NW = NC * NS

EDGES_PER_TILE = N_EDGES // NW
CHUNK = 400
NCHUNK = EDGES_PER_TILE // CHUNK
NGROUP = CHUNK // LANES

NSPLIT = 5
SUB = CHUNK // NSPLIT


def _body(src_hbm, dst_hbm, model_hbm, out_hbm,
          sidx0, didx0, srows0, drows0, outv0, sem0,
          sidx1, didx1, srows1, drows1, outv1, sem1):
    cid = lax.axis_index("c")
    sid = lax.axis_index("s")
    wid = sid * NC + cid
    base = wid * EDGES_PER_TILE

    bufs = ((sidx0, didx0, srows0, drows0, outv0, sem0),
            (sidx1, didx1, srows1, drows1, outv1, sem1))

    def issue(j, b):
        sidx, didx, srows, drows, _, sem = bufs[b]
        off = base + j * CHUNK
        pltpu.sync_copy(src_hbm.at[pl.ds(off, CHUNK)], sidx)
        pltpu.sync_copy(dst_hbm.at[pl.ds(off, CHUNK)], didx)
        for k in range(NSPLIT):
            s = pl.ds(k * SUB, SUB)
            pltpu.async_copy(model_hbm.at[sidx.at[s]], srows.at[s], sem)
            pltpu.async_copy(model_hbm.at[didx.at[s]], drows.at[s], sem)

    def finish(j, b):
        sidx, didx, srows, drows, outv, sem = bufs[b]
        off = base + j * CHUNK
        for k in range(NSPLIT):
            s = pl.ds(k * SUB, SUB)
            pltpu.make_async_copy(model_hbm.at[sidx.at[s]],
                                  srows.at[s], sem).wait()
            pltpu.make_async_copy(model_hbm.at[didx.at[s]],
                                  drows.at[s], sem).wait()

        lane = lax.iota(jnp.int32, LANES)

        def group_step(g, _):
            rows = g * LANES + lane
            acc = jnp.zeros((LANES,), jnp.float32)
            for d in range(D_PACK):
                col = jnp.full((LANES,), d, jnp.int32)
                ws = plsc.bitcast(plsc.load_gather(srows, [rows, col]),
                                  jnp.bfloat16)
                wd = plsc.bitcast(plsc.load_gather(drows, [rows, col]),
                                  jnp.bfloat16)
                s_lo, s_hi = plsc.unpack(ws, format=plsc.PackFormat.INTERLEAVED)
                d_lo, d_hi = plsc.unpack(wd, format=plsc.PackFormat.INTERLEAVED)
                acc += s_lo * d_lo
                acc += s_hi * d_hi
            outv[pl.ds(g * LANES, LANES)] = acc
            return 0

        lax.fori_loop(0, NGROUP, group_step, 0)
        pltpu.sync_copy(outv, out_hbm.at[pl.ds(off, CHUNK)])

    issue(0, 0)

    def pipe_step(i, _):
        j0 = 2 * i
        issue(j0 + 1, 1)
        finish(j0, 0)
        issue(j0 + 2, 0)
        finish(j0 + 1, 1)
        return 0

    lax.fori_loop(0, (NCHUNK - 1) // 2, pipe_step, 0)
    finish(NCHUNK - 1, 0)


@jax.jit
def _run(src, dst, model_packed):
    mesh = plsc.VectorSubcoreMesh(core_axis_name="c", subcore_axis_name="s")
    return pl.kernel(
        _body,
        out_type=jax.ShapeDtypeStruct((N_EDGES,), jnp.float32),
        mesh=mesh,
        compiler_params=pltpu.CompilerParams(needs_layout_passes=False,
                                             use_tc_tiling_on_sc=False),
        scratch_types=[
            pltpu.VMEM((CHUNK,), jnp.int32),
            pltpu.VMEM((CHUNK,), jnp.int32),
            pltpu.VMEM((CHUNK, D_PACK), jnp.int32),
            pltpu.VMEM((CHUNK, D_PACK), jnp.int32),
            pltpu.VMEM((CHUNK,), jnp.float32),
            pltpu.SemaphoreType.DMA,
        ] * 2,
    )(src, dst, model_packed)


def kernel(model, edge_index):
    ei = edge_index.astype(jnp.int32)
    mp = lax.bitcast_convert_type(
        model.astype(jnp.bfloat16).reshape(N_NODES, D_PACK, 2), jnp.int32)
    return _run(ei[0], ei[1], mp)
